# Initial kernel scaffold; baseline (speedup 1.0000x reference)
import functools

import jax
import jax.numpy as jnp
from jax.experimental import pallas as pl
from jax.experimental.pallas import tpu as pltpu
from jax.experimental.pallas import tpu_sc as plsc

_EMBED = 32
_SCALE = float(_EMBED ** 0.5)

_W = 128


def _scale_table(table):
    v, d = table.shape
    flat = table.reshape(v * d // 128, 128)
    rows = flat.shape[0]
    blk = rows
    for cand in (4000, 2000, 1000, 500, 250):
        if rows % cand == 0 and cand % 8 == 0:
            blk = cand
            break

    def body(t_ref, o_ref):
        o_ref[...] = t_ref[...] * _SCALE

    scaled = pl.pallas_call(
        body,
        out_shape=jax.ShapeDtypeStruct(flat.shape, jnp.float32),
        grid=(rows // blk,),
        in_specs=[pl.BlockSpec((blk, 128), lambda i: (i, 0))],
        out_specs=pl.BlockSpec((blk, 128), lambda i: (i, 0)),
    )(flat)
    return scaled.reshape(v, d)


def _gather_rows(scaled_table, idx2d):
    n = idx2d.shape[1]
    mesh = plsc.VectorSubcoreMesh(core_axis_name="core",
                                  subcore_axis_name="subcore")

    @functools.partial(
        pl.kernel,
        out_type=jax.ShapeDtypeStruct((n, _EMBED), jnp.float32),
        mesh=mesh,
    )
    def k(tab_hbm, i_hbm, o_hbm):
        def body(i_vmem, o_vmem):
            pltpu.sync_copy(tab_hbm.at[i_vmem.at[0]], o_vmem)

        pltpu.emit_pipeline(
            body,
            grid=(n // _W,),
            in_specs=[pl.BlockSpec((1, _W), index_map=lambda i: (0, i))],
            out_specs=[pl.BlockSpec((_W, _EMBED), index_map=lambda i: (i, 0))],
            core_axis_name=("core", "subcore"),
            dimension_semantics=(pltpu.PARALLEL,),
        )(i_hbm, o_hbm)

    return k(scaled_table, idx2d)


def kernel(x, table):
    b, s = x.shape
    scaled = _scale_table(table)
    out = _gather_rows(scaled, x.reshape(1, b * s))
    return out.reshape(b, s, _EMBED)


# speedup vs baseline: 5.0682x; 5.0682x over previous
import functools

import jax
import jax.numpy as jnp
from jax.experimental import pallas as pl
from jax.experimental.pallas import tpu as pltpu
from jax.experimental.pallas import tpu_sc as plsc

_EMBED = 32
_SCALE = float(_EMBED ** 0.5)

_W = 128


def _scale_table(table):
    v, d = table.shape
    flat = table.reshape(v * d // 128, 128)
    rows = flat.shape[0]
    blk = rows
    for cand in (4000, 2000, 1000, 500, 250):
        if rows % cand == 0 and cand % 8 == 0:
            blk = cand
            break

    def body(t_ref, o_ref):
        o_ref[...] = t_ref[...] * _SCALE

    scaled = pl.pallas_call(
        body,
        out_shape=jax.ShapeDtypeStruct(flat.shape, jnp.float32),
        grid=(rows // blk,),
        in_specs=[pl.BlockSpec((blk, 128), lambda i: (i, 0))],
        out_specs=pl.BlockSpec((blk, 128), lambda i: (i, 0)),
    )(flat)
    return scaled.reshape(v, d)


def _gather_rows(scaled_table, idx2d):
    n = idx2d.shape[1]
    mesh = plsc.VectorSubcoreMesh(core_axis_name="core",
                                  subcore_axis_name="subcore")

    @functools.partial(
        pl.kernel,
        out_type=jax.ShapeDtypeStruct((n, _EMBED), jnp.float32),
        mesh=mesh,
        compiler_params=pltpu.CompilerParams(use_tc_tiling_on_sc=False),
    )
    def k(tab_hbm, i_hbm, o_hbm):
        def body(i_vmem, o_vmem):
            pltpu.sync_copy(tab_hbm.at[i_vmem.at[0]], o_vmem)

        pltpu.emit_pipeline(
            body,
            grid=(n // _W,),
            in_specs=[pl.BlockSpec((1, _W), index_map=lambda i: (0, i))],
            out_specs=[pl.BlockSpec((_W, _EMBED), index_map=lambda i: (i, 0))],
            core_axis_name=("core", "subcore"),
            dimension_semantics=(pltpu.PARALLEL,),
        )(i_hbm, o_hbm)

    return k(scaled_table, idx2d)


def kernel(x, table):
    b, s = x.shape
    scaled = _scale_table(table)
    out = _gather_rows(scaled, x.reshape(1, b * s))
    return out.reshape(b, s, _EMBED)


# speedup vs baseline: 5.9163x; 1.1673x over previous
import functools

import jax
import jax.numpy as jnp
from jax.experimental import pallas as pl
from jax.experimental.pallas import tpu as pltpu
from jax.experimental.pallas import tpu_sc as plsc

_EMBED = 32
_SCALE = float(_EMBED ** 0.5)

_W = 128

_TB = 2048


def _gather_rows(table, idx2d):
    n = idx2d.shape[1]
    mesh = plsc.VectorSubcoreMesh(core_axis_name="core",
                                  subcore_axis_name="subcore")

    @functools.partial(
        pl.kernel,
        out_type=jax.ShapeDtypeStruct((n, _EMBED), jnp.float32),
        mesh=mesh,
        compiler_params=pltpu.CompilerParams(use_tc_tiling_on_sc=False),
    )
    def k(tab_hbm, i_hbm, o_hbm):
        def body(i_vmem, o_vmem):
            pltpu.sync_copy(tab_hbm.at[i_vmem.at[0]], o_vmem)

        pltpu.emit_pipeline(
            body,
            grid=(n // _W,),
            in_specs=[pl.BlockSpec((1, _W), index_map=lambda i: (0, i))],
            out_specs=[pl.BlockSpec((_W, _EMBED), index_map=lambda i: (i, 0))],
            core_axis_name=("core", "subcore"),
            dimension_semantics=(pltpu.PARALLEL,),
        )(i_hbm, o_hbm)

    return k(table, idx2d)


def _transpose_scale(glin, s, b):
    pack = 128 // _EMBED
    g4 = glin.reshape(s, b * _EMBED // 128, 128)

    chunk = b // pack

    def body(t_ref, o_ref):
        tv = jnp.swapaxes(t_ref[0], 0, 1) * _SCALE
        for a in range(pack):
            o_ref[0, :, a * chunk:(a + 1) * chunk] = \
                tv[a * _EMBED:(a + 1) * _EMBED, :]

    return pl.pallas_call(
        body,
        out_shape=jax.ShapeDtypeStruct((s, _EMBED, b), jnp.float32),
        grid=(s,),
        in_specs=[pl.BlockSpec((1, b * _EMBED // 128, 128),
                               lambda j: (j, 0, 0))],
        out_specs=pl.BlockSpec((1, _EMBED, b), lambda j: (j, 0, 0)),
    )(g4)


def kernel(x, table):
    b, s = x.shape
    n = b * s
    pack = 128 // _EMBED
    xt = jnp.swapaxes(x, 0, 1)
    idx = xt.reshape(s, pack, b // pack).swapaxes(1, 2).reshape(1, n)
    g = _gather_rows(table, idx)
    outp = _transpose_scale(g.reshape(n * _EMBED), s, b)
    return jnp.transpose(outp, (2, 0, 1))


# speedup vs baseline: 9.8631x; 1.6671x over previous
import functools

import jax
import jax.numpy as jnp
from jax.experimental import pallas as pl
from jax.experimental.pallas import tpu as pltpu
from jax.experimental.pallas import tpu_sc as plsc

_EMBED = 32
_SCALE = float(_EMBED ** 0.5)

_W = 128

_TB = 2048


def _gather_rows(table, x4):
    s, pack, nw_j, lanes = x4.shape
    n = s * pack * nw_j * lanes
    w_rows = pack * lanes
    mesh = plsc.VectorSubcoreMesh(core_axis_name="core",
                                  subcore_axis_name="subcore")

    @functools.partial(
        pl.kernel,
        out_type=jax.ShapeDtypeStruct((n, _EMBED), jnp.float32),
        mesh=mesh,
        compiler_params=pltpu.CompilerParams(use_tc_tiling_on_sc=False,
                                             needs_layout_passes=False),
        scratch_types=[
            pltpu.VMEM((w_rows,), jnp.int32),
            pltpu.SemaphoreType.DMA,
        ],
    )
    def k(tab_hbm, i_hbm, o_hbm, idx_s, gsem):
        def body(i_vmem, o_vmem):
            src = i_vmem.at[0, :, 0, :]
            for q in range(w_rows // 16):
                l = jax.lax.iota(jnp.int32, 16)
                row = jax.lax.rem(l, pack)
                col = (pack * q) + jax.lax.div(l, pack)
                idx_s[pl.ds(16 * q, 16)] = plsc.load_gather(src, [row, col])
            copies = [
                pltpu.async_copy(
                    tab_hbm.at[idx_s.at[pl.ds(g * lanes, lanes)]],
                    o_vmem.at[pl.ds(g * lanes, lanes)], gsem)
                for g in range(pack)
            ]
            for c in copies:
                c.wait()

        pltpu.emit_pipeline(
            body,
            grid=(n // w_rows,),
            in_specs=[pl.BlockSpec(
                (1, pack, 1, lanes),
                index_map=lambda w: (w // nw_j, 0, w % nw_j, 0))],
            out_specs=[pl.BlockSpec((w_rows, _EMBED),
                                    index_map=lambda w: (w, 0))],
            core_axis_name=("core", "subcore"),
            dimension_semantics=(pltpu.PARALLEL,),
        )(i_hbm, o_hbm)

    return k(table, x4)


def _transpose_scale(glin, s, b):
    pack = 128 // _EMBED
    g4 = glin.reshape(s, b * _EMBED // 128, 128)

    chunk = b // pack

    def body(t_ref, o_ref):
        tv = jnp.swapaxes(t_ref[0], 0, 1) * _SCALE
        for a in range(pack):
            o_ref[0, :, a * chunk:(a + 1) * chunk] = \
                tv[a * _EMBED:(a + 1) * _EMBED, :]

    return pl.pallas_call(
        body,
        out_shape=jax.ShapeDtypeStruct((s, _EMBED, b), jnp.float32),
        grid=(s,),
        in_specs=[pl.BlockSpec((1, b * _EMBED // 128, 128),
                               lambda j: (j, 0, 0))],
        out_specs=pl.BlockSpec((1, _EMBED, b), lambda j: (j, 0, 0)),
    )(g4)


def kernel(x, table):
    b, s = x.shape
    n = b * s
    pack = 128 // _EMBED
    xt = jnp.swapaxes(x, 0, 1)
    x4 = xt.reshape(s, pack, b // (pack * 128), 128)
    g = _gather_rows(table, x4)
    outp = _transpose_scale(g.reshape(n * _EMBED), s, b)
    return jnp.transpose(outp, (2, 0, 1))
